# baseline (speedup 1.0000x reference)
import functools

import jax
import jax.numpy as jnp
from jax import lax
from jax.experimental import pallas as pl
from jax.experimental.pallas import tpu as pltpu
from jax.experimental.pallas import tpu_sc as plsc

EMB = 64
NC = 2
NS = 16
NW = NC * NS
CHUNK = 128
HALF = CHUNK // 2
LANES = 16
GRP = 8


@functools.lru_cache(maxsize=None)
def _build_gather(N):
    n_per_w = N // NW
    n_chunks = n_per_w // CHUNK
    n_groups = n_chunks // GRP
    p_per_w = n_per_w // 2
    mesh = plsc.VectorSubcoreMesh(core_axis_name="c", subcore_axis_name="s")

    @functools.partial(
        pl.kernel,
        mesh=mesh,
        compiler_params=pltpu.CompilerParams(use_tc_tiling_on_sc=False,
                                             needs_layout_passes=False),
        out_type=jax.ShapeDtypeStruct((N // 2, 2 * EMB), jnp.float32),
        scratch_types=[
            pltpu.VMEM((n_chunks, CHUNK), jnp.int32),
            pltpu.VMEM((n_chunks, CHUNK), jnp.int32),
            pltpu.VMEM((n_chunks, CHUNK), jnp.int32),
            pltpu.VMEM((GRP, CHUNK, EMB), jnp.float32),
            pltpu.SemaphoreType.DMA,
            pltpu.SemaphoreType.DMA,
        ],
    )
    def emb_kernel(labels_hbm, l2g_hbm, table_hbm, x_hbm,
                   idx_v, idx2_v, gidx_v, rows_v, gsem, osem):
        wid = lax.axis_index("s") * NC + lax.axis_index("c")
        p_base = wid * p_per_w

        pltpu.sync_copy(labels_hbm.at[wid], idx_v)

        lane = lax.iota(jnp.int32, LANES)

        def shuf_body(j, carry):
            for v in range(CHUNK // LANES):
                s0 = v * LANES
                if s0 < HALF:
                    src = 2 * s0 + 2 * lane
                else:
                    src = 2 * (s0 - HALF) + 2 * lane + 1
                row = jnp.full((LANES,), j, dtype=jnp.int32)
                vals = plsc.load_gather(idx_v, [row, src])
                idx2_v[j, pl.ds(s0, LANES)] = vals
            return carry

        lax.fori_loop(0, n_chunks, shuf_body, 0, unroll=False)

        def gidx_body(jj, carry):
            handles = []
            for b in range(GRP):
                j = jj * GRP + b
                handles.append(
                    pltpu.async_copy(l2g_hbm.at[idx2_v.at[j]], gidx_v.at[j],
                                     gsem))
            for h in handles:
                h.wait()
            return carry

        lax.fori_loop(0, n_groups, gidx_body, 0, unroll=False)

        def row_body(jj, carry):
            @pl.when(jj > 0)
            def _drain_prev():
                for b in range(GRP):
                    for h in range(2):
                        pltpu.make_async_copy(
                            rows_v.at[b, pl.ds(h * HALF, HALF)],
                            x_hbm.at[pl.ds(p_base, HALF),
                                     pl.ds(h * EMB, EMB)],
                            osem).wait()

            handles = []
            for b in range(GRP):
                j = jj * GRP + b
                handles.append(
                    pltpu.async_copy(table_hbm.at[gidx_v.at[j]], rows_v.at[b],
                                     gsem))
            for b in range(GRP):
                j = jj * GRP + b
                handles[b].wait()
                p0 = p_base + j * HALF
                for h in range(2):
                    pltpu.async_copy(
                        rows_v.at[b, pl.ds(h * HALF, HALF)],
                        x_hbm.at[pl.ds(p0, HALF), pl.ds(h * EMB, EMB)],
                        osem)
            return carry

        lax.fori_loop(0, n_groups, row_body, 0, unroll=False)
        for b in range(GRP):
            for h in range(2):
                pltpu.make_async_copy(
                    rows_v.at[b, pl.ds(h * HALF, HALF)],
                    x_hbm.at[pl.ds(p_base, HALF), pl.ds(h * EMB, EMB)],
                    osem).wait()

    return emb_kernel


BB = 128


@functools.lru_cache(maxsize=None)
def _build_retile(B, L):
    def body(x_ref, o_ref):
        x = x_ref[...]
        a = x[:, :EMB]
        b = x[:, EMB:]
        y = jnp.stack([a, b], axis=1)
        o_ref[...] = y.reshape(BB, L, EMB)

    return pl.pallas_call(
        body,
        grid=(B // BB,),
        in_specs=[pl.BlockSpec((BB * L // 2, 2 * EMB), lambda i: (i, 0))],
        out_specs=pl.BlockSpec((BB, L, EMB), lambda i: (i, 0, 0)),
        out_shape=jax.ShapeDtypeStruct((B, L, EMB), jnp.float32),
    )


def kernel(label_ids, local2global, table):
    B, L = label_ids.shape
    N = B * L
    labels = label_ids.reshape(NW, N // NW // CHUNK, CHUNK)
    x = _build_gather(N)(labels, local2global, table)
    return _build_retile(B, L)(x)


# speedup vs baseline: 1.7053x; 1.7053x over previous
import functools

import jax
import jax.numpy as jnp
from jax import lax
from jax.experimental import pallas as pl
from jax.experimental.pallas import tpu as pltpu
from jax.experimental.pallas import tpu_sc as plsc

EMB = 64
NC = 2
NS = 16
NW = NC * NS
CHUNK = 128
HALF = CHUNK // 2
LANES = 16
GRP = 8


@functools.lru_cache(maxsize=None)
def _build_gather(N):
    n_per_w = N // NW
    n_chunks = n_per_w // CHUNK
    n_groups = n_chunks // GRP
    p_per_w = n_per_w // 2
    mesh = plsc.VectorSubcoreMesh(core_axis_name="c", subcore_axis_name="s")

    @functools.partial(
        pl.kernel,
        mesh=mesh,
        compiler_params=pltpu.CompilerParams(use_tc_tiling_on_sc=False,
                                             needs_layout_passes=False),
        out_type=jax.ShapeDtypeStruct((N // 2, 2 * EMB), jnp.float32),
        scratch_types=[
            pltpu.VMEM((n_chunks, CHUNK), jnp.int32),
            pltpu.VMEM((n_chunks, CHUNK), jnp.int32),
            pltpu.VMEM((n_chunks, CHUNK), jnp.int32),
            pltpu.VMEM((GRP, CHUNK, EMB), jnp.float32),
            pltpu.SemaphoreType.DMA,
            pltpu.SemaphoreType.DMA,
        ],
    )
    def emb_kernel(labels_hbm, l2g_hbm, table_hbm, x_hbm,
                   idx_v, idx2_v, gidx_v, rows_v, gsem, osem):
        wid = lax.axis_index("s") * NC + lax.axis_index("c")
        p_base = wid * p_per_w

        pltpu.sync_copy(labels_hbm.at[wid], idx_v)

        lane = lax.iota(jnp.int32, LANES)

        def shuf_body(j, carry):
            for v in range(CHUNK // LANES):
                s0 = v * LANES
                if s0 < HALF:
                    src = 2 * s0 + 2 * lane
                else:
                    src = 2 * (s0 - HALF) + 2 * lane + 1
                row = jnp.full((LANES,), j, dtype=jnp.int32)
                vals = plsc.load_gather(idx_v, [row, src])
                idx2_v[j, pl.ds(s0, LANES)] = vals
            return carry

        lax.fori_loop(0, n_chunks, shuf_body, 0, unroll=False)

        def gidx_body(jj, carry):
            handles = []
            for b in range(GRP):
                j = jj * GRP + b
                handles.append(
                    pltpu.async_copy(l2g_hbm.at[idx2_v.at[j]], gidx_v.at[j],
                                     gsem))
            for h in handles:
                h.wait()
            return carry

        lax.fori_loop(0, n_groups, gidx_body, 0, unroll=False)

        def row_body(jj, carry):
            @pl.when(jj > 0)
            def _drain_prev():
                for b in range(GRP):
                    for h in range(2):
                        pltpu.make_async_copy(
                            rows_v.at[b, pl.ds(h * HALF, HALF)],
                            x_hbm.at[pl.ds(p_base, HALF),
                                     pl.ds(h * EMB, EMB)],
                            osem).wait()

            handles = []
            for b in range(GRP):
                j = jj * GRP + b
                handles.append(
                    pltpu.async_copy(table_hbm.at[gidx_v.at[j]], rows_v.at[b],
                                     gsem))
            for b in range(GRP):
                j = jj * GRP + b
                handles[b].wait()
                p0 = p_base + j * HALF
                for h in range(2):
                    pltpu.async_copy(
                        rows_v.at[b, pl.ds(h * HALF, HALF)],
                        x_hbm.at[pl.ds(p0, HALF), pl.ds(h * EMB, EMB)],
                        osem)
            return carry

        lax.fori_loop(0, n_groups, row_body, 0, unroll=False)
        for b in range(GRP):
            for h in range(2):
                pltpu.make_async_copy(
                    rows_v.at[b, pl.ds(h * HALF, HALF)],
                    x_hbm.at[pl.ds(p_base, HALF), pl.ds(h * EMB, EMB)],
                    osem).wait()

    return emb_kernel


BB = 128


@functools.lru_cache(maxsize=None)
def _build_retile(B, L):
    LH = L // 2

    def body(x_ref, o_ref):
        x3 = x_ref[...].reshape(BB, LH, 2 * EMB)
        for w in range(LH):
            t = x3[:, w, :].T
            o_ref[2 * w, :, :] = t[:EMB, :]
            o_ref[2 * w + 1, :, :] = t[EMB:, :]

    return pl.pallas_call(
        body,
        grid=(B // BB,),
        in_specs=[pl.BlockSpec((BB * LH, 2 * EMB), lambda i: (i, 0))],
        out_specs=pl.BlockSpec((L, EMB, BB), lambda i: (0, 0, i)),
        out_shape=jax.ShapeDtypeStruct((L, EMB, B), jnp.float32),
    )


def kernel(label_ids, local2global, table):
    B, L = label_ids.shape
    N = B * L
    labels = label_ids.reshape(NW, N // NW // CHUNK, CHUNK)
    x = _build_gather(N)(labels, local2global, table)
    p = _build_retile(B, L)(x)
    return jnp.transpose(p, (2, 0, 1))


# speedup vs baseline: 2.0319x; 1.1915x over previous
import functools

import jax
import jax.numpy as jnp
from jax import lax
from jax.experimental import pallas as pl
from jax.experimental.pallas import tpu as pltpu
from jax.experimental.pallas import tpu_sc as plsc

EMB = 64
NC = 2
NS = 16
NW = NC * NS
CHUNK = 128
HALF = CHUNK // 2
LANES = 16
GRP = 8

_SC_PARAMS = pltpu.CompilerParams(use_tc_tiling_on_sc=False,
                                  needs_layout_passes=False)
_MESH = plsc.VectorSubcoreMesh(core_axis_name="c", subcore_axis_name="s")


@functools.lru_cache(maxsize=None)
def _build_k1(N):
    n_per_w = N // NW
    n_chunks = n_per_w // CHUNK
    n_groups = n_chunks // GRP

    @functools.partial(
        pl.kernel,
        mesh=_MESH,
        compiler_params=_SC_PARAMS,
        out_type=jax.ShapeDtypeStruct((NW, n_chunks, CHUNK), jnp.int32),
        scratch_types=[
            pltpu.VMEM((n_chunks, CHUNK), jnp.int32),
            pltpu.VMEM((n_chunks, CHUNK), jnp.int32),
            pltpu.VMEM((n_chunks, CHUNK), jnp.int32),
            pltpu.SemaphoreType.DMA,
        ],
    )
    def k1(labels_hbm, l2g_hbm, gout_hbm, idx_v, idx2_v, gidx_v, gsem):
        wid = lax.axis_index("s") * NC + lax.axis_index("c")
        pltpu.sync_copy(labels_hbm.at[wid], idx_v)

        lane = lax.iota(jnp.int32, LANES)

        def shuf_body(j, carry):
            for v in range(CHUNK // LANES):
                s0 = v * LANES
                if s0 < HALF:
                    src = 2 * s0 + 2 * lane
                else:
                    src = 2 * (s0 - HALF) + 2 * lane + 1
                row = jnp.full((LANES,), j, dtype=jnp.int32)
                vals = plsc.load_gather(idx_v, [row, src])
                idx2_v[j, pl.ds(s0, LANES)] = vals
            return carry

        lax.fori_loop(0, n_chunks, shuf_body, 0, unroll=False)

        def gidx_body(jj, carry):
            handles = []
            for b in range(GRP):
                j = jj * GRP + b
                handles.append(
                    pltpu.async_copy(l2g_hbm.at[idx2_v.at[j]], gidx_v.at[j],
                                     gsem))
            for h in handles:
                h.wait()
            return carry

        lax.fori_loop(0, n_groups, gidx_body, 0, unroll=False)
        pltpu.sync_copy(gidx_v, gout_hbm.at[wid])

    return k1


@functools.lru_cache(maxsize=None)
def _build_k2(N):
    n_per_w = N // NW
    n_chunks = n_per_w // CHUNK
    n_groups = n_chunks // GRP
    p_per_w = n_per_w // 2

    @functools.partial(
        pl.kernel,
        mesh=_MESH,
        compiler_params=_SC_PARAMS,
        out_type=jax.ShapeDtypeStruct((N // 2, 2 * EMB), jnp.float32),
        scratch_types=[
            pltpu.VMEM((n_chunks, CHUNK), jnp.int32),
            pltpu.VMEM((GRP, CHUNK, EMB), jnp.float32),
            pltpu.SemaphoreType.DMA,
            pltpu.SemaphoreType.DMA,
        ],
    )
    def k2(gidx_hbm, table_hbm, x_hbm, gidx_v, rows_v, gsem, osem):
        wid = lax.axis_index("s") * NC + lax.axis_index("c")
        p_base = wid * p_per_w
        pltpu.sync_copy(gidx_hbm.at[wid], gidx_v)

        def row_body(jj, carry):
            @pl.when(jj > 0)
            def _drain_prev():
                for b in range(GRP):
                    for h in range(2):
                        pltpu.make_async_copy(
                            rows_v.at[b, pl.ds(h * HALF, HALF)],
                            x_hbm.at[pl.ds(p_base, HALF),
                                     pl.ds(h * EMB, EMB)],
                            osem).wait()

            handles = []
            for b in range(GRP):
                j = jj * GRP + b
                handles.append(
                    pltpu.async_copy(table_hbm.at[gidx_v.at[j]], rows_v.at[b],
                                     gsem))
            for b in range(GRP):
                j = jj * GRP + b
                handles[b].wait()
                p0 = p_base + j * HALF
                for h in range(2):
                    pltpu.async_copy(
                        rows_v.at[b, pl.ds(h * HALF, HALF)],
                        x_hbm.at[pl.ds(p0, HALF), pl.ds(h * EMB, EMB)],
                        osem)
            return carry

        lax.fori_loop(0, n_groups, row_body, 0, unroll=False)
        for b in range(GRP):
            for h in range(2):
                pltpu.make_async_copy(
                    rows_v.at[b, pl.ds(h * HALF, HALF)],
                    x_hbm.at[pl.ds(p_base, HALF), pl.ds(h * EMB, EMB)],
                    osem).wait()

    return k2


BB = 256


@functools.lru_cache(maxsize=None)
def _build_retile(B, L):
    LH = L // 2

    def body(x_ref, o_ref):
        x3 = x_ref[...].reshape(BB, LH, 2 * EMB)
        for w in range(LH):
            t = x3[:, w, :].T
            o_ref[2 * w, :, :] = t[:EMB, :]
            o_ref[2 * w + 1, :, :] = t[EMB:, :]

    return pl.pallas_call(
        body,
        grid=(B // BB,),
        in_specs=[pl.BlockSpec((BB * LH, 2 * EMB), lambda i: (i, 0))],
        out_specs=pl.BlockSpec((L, EMB, BB), lambda i: (0, 0, i)),
        out_shape=jax.ShapeDtypeStruct((L, EMB, B), jnp.float32),
    )


def kernel(label_ids, local2global, table):
    B, L = label_ids.shape
    N = B * L
    labels = label_ids.reshape(NW, N // NW // CHUNK, CHUNK)
    gidx = _build_k1(N)(labels, local2global)
    x = _build_k2(N)(gidx, table)
    p = _build_retile(B, L)(x)
    return jnp.transpose(p, (2, 0, 1))


# speedup vs baseline: 2.1096x; 1.0382x over previous
import functools

import jax
import jax.numpy as jnp
from jax import lax
from jax.experimental import pallas as pl
from jax.experimental.pallas import tpu as pltpu
from jax.experimental.pallas import tpu_sc as plsc

EMB = 64
NC = 2
NS = 16
NW = NC * NS
CHUNK = 128
HALF = CHUNK // 2
LANES = 16
GRP = 8

_SC_PARAMS = pltpu.CompilerParams(use_tc_tiling_on_sc=False,
                                  needs_layout_passes=False)
_MESH = plsc.VectorSubcoreMesh(core_axis_name="c", subcore_axis_name="s")


@functools.lru_cache(maxsize=None)
def _build_k1(N):
    n_per_w = N // NW
    n_chunks = n_per_w // CHUNK
    n_groups = n_chunks // GRP

    @functools.partial(
        pl.kernel,
        mesh=_MESH,
        compiler_params=_SC_PARAMS,
        out_type=jax.ShapeDtypeStruct((NW, n_chunks, CHUNK), jnp.int32),
        scratch_types=[
            pltpu.VMEM((n_chunks, CHUNK), jnp.int32),
            pltpu.VMEM((n_chunks, CHUNK), jnp.int32),
            pltpu.VMEM((n_chunks, CHUNK), jnp.int32),
            pltpu.SemaphoreType.DMA,
        ],
    )
    def k1(labels_hbm, l2g_hbm, gout_hbm, idx_v, idx2_v, gidx_v, gsem):
        wid = lax.axis_index("s") * NC + lax.axis_index("c")
        pltpu.sync_copy(labels_hbm.at[wid], idx_v)

        lane = lax.iota(jnp.int32, LANES)

        def shuf_body(j, carry):
            for v in range(CHUNK // LANES):
                s0 = v * LANES
                if s0 < HALF:
                    src = 2 * s0 + 2 * lane
                else:
                    src = 2 * (s0 - HALF) + 2 * lane + 1
                row = jnp.full((LANES,), j, dtype=jnp.int32)
                vals = plsc.load_gather(idx_v, [row, src])
                idx2_v[j, pl.ds(s0, LANES)] = vals
            return carry

        lax.fori_loop(0, n_chunks, shuf_body, 0, unroll=False)

        def gidx_body(jj, carry):
            handles = []
            for b in range(GRP):
                j = jj * GRP + b
                handles.append(
                    pltpu.async_copy(l2g_hbm.at[idx2_v.at[j]], gidx_v.at[j],
                                     gsem))
            for h in handles:
                h.wait()
            return carry

        lax.fori_loop(0, n_groups, gidx_body, 0, unroll=False)
        pltpu.sync_copy(gidx_v, gout_hbm.at[wid])

    return k1


NSPLIT = 2


@functools.lru_cache(maxsize=None)
def _build_k2(N, half_idx):
    n_per_w = N // NW
    n_chunks = n_per_w // CHUNK // NSPLIT
    n_groups = n_chunks // GRP
    p_per_w = n_chunks * HALF

    @functools.partial(
        pl.kernel,
        mesh=_MESH,
        compiler_params=_SC_PARAMS,
        out_type=jax.ShapeDtypeStruct((N // 2 // NSPLIT, 2 * EMB),
                                      jnp.float32),
        scratch_types=[
            pltpu.VMEM((n_chunks, CHUNK), jnp.int32),
            pltpu.VMEM((GRP, CHUNK, EMB), jnp.float32),
            pltpu.SemaphoreType.DMA,
            pltpu.SemaphoreType.DMA,
        ],
    )
    def k2(gidx_hbm, table_hbm, x_hbm, gidx_v, rows_v, gsem, osem):
        wid = lax.axis_index("s") * NC + lax.axis_index("c")
        p_base = wid * p_per_w
        pltpu.sync_copy(gidx_hbm.at[wid, pl.ds(half_idx * n_chunks, n_chunks)],
                        gidx_v)

        def row_body(jj, carry):
            @pl.when(jj > 0)
            def _drain_prev():
                for b in range(GRP):
                    for h in range(2):
                        pltpu.make_async_copy(
                            rows_v.at[b, pl.ds(h * HALF, HALF)],
                            x_hbm.at[pl.ds(p_base, HALF),
                                     pl.ds(h * EMB, EMB)],
                            osem).wait()

            handles = []
            for b in range(GRP):
                j = jj * GRP + b
                handles.append(
                    pltpu.async_copy(table_hbm.at[gidx_v.at[j]], rows_v.at[b],
                                     gsem))
            for b in range(GRP):
                j = jj * GRP + b
                handles[b].wait()
                p0 = p_base + j * HALF
                for h in range(2):
                    pltpu.async_copy(
                        rows_v.at[b, pl.ds(h * HALF, HALF)],
                        x_hbm.at[pl.ds(p0, HALF), pl.ds(h * EMB, EMB)],
                        osem)
            return carry

        lax.fori_loop(0, n_groups, row_body, 0, unroll=False)
        for b in range(GRP):
            for h in range(2):
                pltpu.make_async_copy(
                    rows_v.at[b, pl.ds(h * HALF, HALF)],
                    x_hbm.at[pl.ds(p_base, HALF), pl.ds(h * EMB, EMB)],
                    osem).wait()

    return k2


BB = 256


@functools.lru_cache(maxsize=None)
def _build_retile(B, L, h):
    LH = L // 2
    b_per_w = B // NW

    def body(x_ref, *rest):
        o_ref = rest[-1]
        x3 = x_ref[...].reshape(BB, LH, 2 * EMB)
        for w in range(LH):
            t = x3[:, w, :].T
            o_ref[2 * w, :, :] = t[:EMB, :]
            o_ref[2 * w + 1, :, :] = t[EMB:, :]

    in_specs = [pl.BlockSpec((BB * LH, 2 * EMB), lambda i: (i, 0))]
    if h == 1:
        in_specs.append(pl.BlockSpec(memory_space=pl.ANY))

    return pl.pallas_call(
        body,
        grid=(NW,),
        in_specs=in_specs,
        out_specs=pl.BlockSpec(
            (L, EMB, BB),
            lambda i: (0, 0, i * (b_per_w // BB) + h)),
        out_shape=jax.ShapeDtypeStruct((L, EMB, B), jnp.float32),
        input_output_aliases={1: 0} if h == 1 else {},
    )


def kernel(label_ids, local2global, table):
    B, L = label_ids.shape
    N = B * L
    labels = label_ids.reshape(NW, N // NW // CHUNK, CHUNK)
    gidx = _build_k1(N)(labels, local2global)
    xa = _build_k2(N, 0)(gidx, table)
    xb = _build_k2(N, 1)(gidx, table)
    pa = _build_retile(B, L, 0)(xa)
    p = _build_retile(B, L, 1)(xb, pa)
    return jnp.transpose(p, (2, 0, 1))


# speedup vs baseline: 2.2494x; 1.0663x over previous
import functools

import jax
import jax.numpy as jnp
from jax import lax
from jax.experimental import pallas as pl
from jax.experimental.pallas import tpu as pltpu
from jax.experimental.pallas import tpu_sc as plsc

EMB = 64
NC = 2
NS = 16
NW = NC * NS
CHUNK = 128
HALF = CHUNK // 2
LANES = 16
GRP = 8

_SC_PARAMS = pltpu.CompilerParams(use_tc_tiling_on_sc=False,
                                  needs_layout_passes=False)
_MESH = plsc.VectorSubcoreMesh(core_axis_name="c", subcore_axis_name="s")


@functools.lru_cache(maxsize=None)
def _build_k1(N):
    n_per_w = N // NW
    n_chunks = n_per_w // CHUNK
    n_groups = n_chunks // GRP

    @functools.partial(
        pl.kernel,
        mesh=_MESH,
        compiler_params=_SC_PARAMS,
        out_type=jax.ShapeDtypeStruct((NW, n_chunks, CHUNK), jnp.int32),
        scratch_types=[
            pltpu.VMEM((n_chunks, CHUNK), jnp.int32),
            pltpu.VMEM((n_chunks, CHUNK), jnp.int32),
            pltpu.VMEM((n_chunks, CHUNK), jnp.int32),
            pltpu.SemaphoreType.DMA,
        ],
    )
    def k1(labels_hbm, l2g_hbm, gout_hbm, idx_v, idx2_v, gidx_v, gsem):
        wid = lax.axis_index("s") * NC + lax.axis_index("c")
        pltpu.sync_copy(labels_hbm.at[wid], idx_v)

        lane = lax.iota(jnp.int32, LANES)

        def shuf_body(j, carry):
            for v in range(CHUNK // LANES):
                s0 = v * LANES
                if s0 < HALF:
                    src = 2 * s0 + 2 * lane
                else:
                    src = 2 * (s0 - HALF) + 2 * lane + 1
                row = jnp.full((LANES,), j, dtype=jnp.int32)
                vals = plsc.load_gather(idx_v, [row, src])
                idx2_v[j, pl.ds(s0, LANES)] = vals
            return carry

        lax.fori_loop(0, n_chunks, shuf_body, 0, unroll=False)

        def gidx_body(jj, carry):
            handles = []
            for b in range(GRP):
                j = jj * GRP + b
                handles.append(
                    pltpu.async_copy(l2g_hbm.at[idx2_v.at[j]], gidx_v.at[j],
                                     gsem))
            for h in handles:
                h.wait()
            return carry

        lax.fori_loop(0, n_groups, gidx_body, 0, unroll=False)
        pltpu.sync_copy(gidx_v, gout_hbm.at[wid])

    return k1


NSPLIT = 2


@functools.lru_cache(maxsize=None)
def _build_k2(N, half_idx):
    n_per_w = N // NW
    n_chunks = n_per_w // CHUNK // NSPLIT
    n_groups = n_chunks // GRP
    p_per_w = n_chunks * HALF

    @functools.partial(
        pl.kernel,
        mesh=_MESH,
        compiler_params=_SC_PARAMS,
        out_type=jax.ShapeDtypeStruct((N // 2 // NSPLIT, 2 * EMB),
                                      jnp.float32),
        scratch_types=[
            pltpu.VMEM((n_chunks, CHUNK), jnp.int32),
            pltpu.VMEM((GRP, CHUNK, EMB), jnp.float32),
            pltpu.SemaphoreType.DMA,
            pltpu.SemaphoreType.DMA,
        ],
    )
    def k2(gidx_hbm, table_hbm, x_hbm, gidx_v, rows_v, gsem, osem):
        wid = lax.axis_index("s") * NC + lax.axis_index("c")
        p_base = wid * p_per_w
        pltpu.sync_copy(gidx_hbm.at[wid, pl.ds(half_idx * n_chunks, n_chunks)],
                        gidx_v)

        def row_body(jj, carry):
            @pl.when(jj > 0)
            def _drain_prev():
                for b in range(GRP):
                    for h in range(2):
                        pltpu.make_async_copy(
                            rows_v.at[b, pl.ds(h * HALF, HALF)],
                            x_hbm.at[pl.ds(p_base, HALF),
                                     pl.ds(h * EMB, EMB)],
                            osem).wait()

            handles = []
            for b in range(GRP):
                j = jj * GRP + b
                handles.append(
                    pltpu.async_copy(table_hbm.at[gidx_v.at[j]], rows_v.at[b],
                                     gsem))
            for b in range(GRP):
                j = jj * GRP + b
                handles[b].wait()
                p0 = p_base + j * HALF
                for h in range(2):
                    pltpu.async_copy(
                        rows_v.at[b, pl.ds(h * HALF, HALF)],
                        x_hbm.at[pl.ds(p0, HALF), pl.ds(h * EMB, EMB)],
                        osem)
            return carry

        lax.fori_loop(0, n_groups, row_body, 0, unroll=False)
        for b in range(GRP):
            for h in range(2):
                pltpu.make_async_copy(
                    rows_v.at[b, pl.ds(h * HALF, HALF)],
                    x_hbm.at[pl.ds(p_base, HALF), pl.ds(h * EMB, EMB)],
                    osem).wait()

    return k2


BV = 12800


@functools.lru_cache(maxsize=None)
def _build_tableconv(V):
    def body(tv_ref, y_ref):
        t = tv_ref[...].T
        t2 = t.reshape(BV // 2, 2, EMB)
        y_ref[:, :EMB] = t2[:, 0, :]
        y_ref[:, EMB:] = t2[:, 1, :]

    return pl.pallas_call(
        body,
        grid=(pl.cdiv(V, BV),),
        in_specs=[pl.BlockSpec((EMB, BV), lambda i: (0, i))],
        out_specs=pl.BlockSpec((BV // 2, 2 * EMB), lambda i: (i, 0)),
        out_shape=jax.ShapeDtypeStruct((V // 2, 2 * EMB), jnp.float32),
    )


BB = 256


@functools.lru_cache(maxsize=None)
def _build_retile(B, L, h):
    LH = L // 2
    b_per_w = B // NW

    def body(x_ref, *rest):
        o_ref = rest[-1]
        x3 = x_ref[...].reshape(BB, LH, 2 * EMB)
        for w in range(LH):
            t = x3[:, w, :].T
            o_ref[2 * w, :, :] = t[:EMB, :]
            o_ref[2 * w + 1, :, :] = t[EMB:, :]

    in_specs = [pl.BlockSpec((BB * LH, 2 * EMB), lambda i: (i, 0))]
    if h == 1:
        in_specs.append(pl.BlockSpec(memory_space=pl.ANY))

    return pl.pallas_call(
        body,
        grid=(NW,),
        in_specs=in_specs,
        out_specs=pl.BlockSpec(
            (L, EMB, BB),
            lambda i: (0, 0, i * (b_per_w // BB) + h)),
        out_shape=jax.ShapeDtypeStruct((L, EMB, B), jnp.float32),
        input_output_aliases={1: 0} if h == 1 else {},
    )


def kernel(label_ids, local2global, table):
    B, L = label_ids.shape
    N = B * L
    labels = label_ids.reshape(NW, N // NW // CHUNK, CHUNK)
    V = table.shape[0]
    table_u = _build_tableconv(V)(table.T).reshape(V, EMB)
    gidx = _build_k1(N)(labels, local2global)
    xa = _build_k2(N, 0)(gidx, table_u)
    xb = _build_k2(N, 1)(gidx, table_u)
    pa = _build_retile(B, L, 0)(xa)
    p = _build_retile(B, L, 1)(xb, pa)
    return jnp.transpose(p, (2, 0, 1))
